# Initial kernel scaffold; baseline (speedup 1.0000x reference)
import jax
import jax.numpy as jnp
from jax.experimental import pallas as pl


def kernel(x, obs_x_index, edge_index_01, edge_index_2, W_obs0, b_obs0, W_obs1, b_obs1, W_obs2, b_obs2, W_q0, b_q0, W_q1, b_q1, W_q2, b_q2, W_v0, b_v0, W_v1, b_v1, W_v2, b_v2, W_bil, b_bil, W_g, b_g):
    raise NotImplementedError("write your pallas kernel here")


# speedup vs baseline: 1.3164x; 1.3164x over previous
import functools
import math

import jax
import jax.numpy as jnp
from jax import lax
from jax.experimental import pallas as pl
from jax.experimental.pallas import tpu as pltpu
from jax.experimental.pallas import tpu_sc as plsc

_BF = jnp.bfloat16
_F32 = jnp.float32


def _sc_gather(x, idx_pad):
    b, h = idx_pad.shape[0], x.shape[1]
    info = plsc.get_sparse_core_info()
    nw = info.num_cores * info.num_subcores
    b_per_w = b // nw
    mesh = plsc.VectorSubcoreMesh(core_axis_name="c", subcore_axis_name="s")

    @functools.partial(
        pl.kernel,
        mesh=mesh,
        out_type=jax.ShapeDtypeStruct((b, h), _F32),
        scratch_types=[
            pltpu.VMEM((b_per_w,), jnp.int32),
            pltpu.VMEM((b_per_w, h), _F32),
            pltpu.SemaphoreType.DMA,
        ],
    )
    def gather_kernel(x_hbm, idx_hbm, out_hbm, idx_v, rows_v, sem):
        wid = lax.axis_index("s") * info.num_cores + lax.axis_index("c")
        base = wid * b_per_w
        pltpu.sync_copy(idx_hbm.at[pl.ds(base, b_per_w)], idx_v)
        pltpu.async_copy(x_hbm.at[idx_v], rows_v, sem).wait()
        pltpu.sync_copy(rows_v, out_hbm.at[pl.ds(base, b_per_w)])

    return gather_kernel(x, idx_pad)


def _mlp3(z, w_refs, b_refs):
    for w_ref, b_ref in zip(w_refs, b_refs):
        w = w_ref[...].astype(_BF)
        z = jnp.dot(z, w, preferred_element_type=_F32) + b_ref[...]
        z = jnp.maximum(z, 0.0).astype(_BF)
    return z


def _obs_prep(x_obs, w0, b0, w1, b1, w2, b2, w_bil, ko):
    kop, h = x_obs.shape

    def body(xo_ref, w0r, b0r, w1r, b1r, w2r, b2r, wbil_ref, gw_ref):
        xo = xo_ref[...].astype(_BF)
        hh = _mlp3(xo, (w0r, w1r, w2r), (b0r, b1r, b2r)).astype(_F32)
        rowmask = lax.broadcasted_iota(jnp.int32, (kop, 1), 0) < ko
        g = jnp.sum(jnp.where(rowmask, hh, 0.0), axis=0, keepdims=True) / ko
        gb = g.astype(_BF)
        gw0 = jnp.dot(gb, wbil_ref[0].astype(_BF), preferred_element_type=_F32)
        gw1 = jnp.dot(gb, wbil_ref[1].astype(_BF), preferred_element_type=_F32)
        gw_ref[...] = jnp.concatenate([gw0, gw1], axis=0)

    return pl.pallas_call(
        body,
        out_shape=jax.ShapeDtypeStruct((2, h), _F32),
    )(x_obs, w0, b0, w1, b1, w2, b2, w_bil)


def _body(x, wq, bq, wv, bv, gwt, b_bil, blk):
    n, h = x.shape
    grid = (n + blk - 1) // blk
    npad = grid * blk

    def body(x_ref, wq0, wq1, wq2, bq0, bq1, bq2,
             wv0, wv1, wv2, bv0, bv1, bv2, gwt_ref, bbil_ref,
             dec_ref, v_ref, s_ref):
        i = pl.program_id(0)
        xb = x_ref[...].astype(_BF)
        q = _mlp3(xb, (wq0, wq1, wq2), (bq0, bq1, bq2))
        v = _mlp3(xb, (wv0, wv1, wv2), (bv0, bv1, bv2))
        dec = jnp.dot(q, gwt_ref[...].astype(_BF),
                      preferred_element_type=_F32) + bbil_ref[...]
        dec_ref[...] = dec
        row = i * blk + lax.broadcasted_iota(jnp.int32, (blk, 1), 0)
        valid = row < n
        s_ref[...] = jnp.where(valid, dec[:, 0:1], -jnp.inf)
        v_ref[...] = jnp.where(valid, v, jnp.bfloat16(0.0))

    const = lambda i: (0, 0)
    wspec = pl.BlockSpec((h, h), const)
    bspec = pl.BlockSpec((1, h), const)
    return pl.pallas_call(
        body,
        grid=(grid,),
        in_specs=[
            pl.BlockSpec((blk, h), lambda i: (i, 0)),
            wspec, wspec, wspec, bspec, bspec, bspec,
            wspec, wspec, wspec, bspec, bspec, bspec,
            pl.BlockSpec((h, 2), const),
            pl.BlockSpec((1, 2), const),
        ],
        out_specs=[
            pl.BlockSpec((blk, 2), lambda i: (i, 0)),
            pl.BlockSpec((blk, h), lambda i: (i, 0)),
            pl.BlockSpec((blk, 1), lambda i: (i, 0)),
        ],
        out_shape=[
            jax.ShapeDtypeStruct((n, 2), _F32),
            jax.ShapeDtypeStruct((npad, h), _BF),
            jax.ShapeDtypeStruct((npad, 1), _F32),
        ],
    )(x, wq[0], wq[1], wq[2], bq[0], bq[1], bq[2],
      wv[0], wv[1], wv[2], bv[0], bv[1], bv[2], gwt, b_bil)


def _select_pool(score_mat, score_col, vmat, w_g, b_g, k_pool):
    npad, h = vmat.shape
    nc = b_g.shape[1]

    def body(smat_ref, scol_ref, v_ref, wg_ref, bg_ref, pooled_ref, log_ref):
        big = jnp.uint32(0x80000000)
        sm = smat_ref[...]
        u = lax.bitcast_convert_type(sm, jnp.uint32)
        key = jnp.where(u >= big, ~u, u | big)

        def tstep(i, prefix):
            cand = prefix | lax.shift_right_logical(big, i.astype(jnp.uint32))
            cnt = jnp.sum((key >= cand).astype(jnp.int32))
            return lax.select(cnt >= k_pool, cand, prefix)

        tkey = lax.fori_loop(0, 32, tstep, jnp.uint32(0))

        n_gt = jnp.sum((key > tkey).astype(jnp.int32))
        r = k_pool - n_gt
        rows, cols = sm.shape
        idxm = (lax.broadcasted_iota(jnp.int32, (rows, cols), 0) * cols
                + lax.broadcasted_iota(jnp.int32, (rows, cols), 1))
        tie = key == tkey

        def istep(i, p2):
            cand = p2 | lax.shift_right_logical(jnp.int32(1 << 14), i)
            cnt = jnp.sum((tie & (idxm < cand)).astype(jnp.int32))
            return lax.select(cnt < r, cand, p2)

        limit = lax.fori_loop(0, 15, istep, jnp.int32(0)) + 1

        m = jnp.max(sm)
        sc = scol_ref[...]
        uc = lax.bitcast_convert_type(sc, jnp.uint32)
        keyc = jnp.where(uc >= big, ~uc, uc | big)
        idxc = lax.broadcasted_iota(jnp.int32, sc.shape, 0)
        sel = (keyc > tkey) | ((keyc == tkey) & (idxc < limit))
        e = jnp.where(sel, jnp.exp(sc - m), 0.0)
        z = jnp.sum(e)
        vv = v_ref[...].astype(_F32)
        pooled = jnp.sum(e * vv, axis=0, keepdims=True) / z
        pooled_ref[...] = pooled
        lg = jnp.dot(pooled.astype(_BF), wg_ref[...].astype(_BF),
                     preferred_element_type=_F32) + bg_ref[...]
        log_ref[...] = lg

    return pl.pallas_call(
        body,
        out_shape=[
            jax.ShapeDtypeStruct((1, h), _F32),
            jax.ShapeDtypeStruct((1, nc), _F32),
        ],
    )(score_mat, score_col, vmat, w_g, b_g)


def kernel(x, obs_x_index, edge_index_01, edge_index_2,
           W_obs0, b_obs0, W_obs1, b_obs1, W_obs2, b_obs2,
           W_q0, b_q0, W_q1, b_q1, W_q2, b_q2,
           W_v0, b_v0, W_v1, b_v1, W_v2, b_v2,
           W_bil, b_bil, W_g, b_g):
    n, h = x.shape
    ko = obs_x_index.shape[0]
    kop = ((ko + 255) // 256) * 256
    k_pool = int(math.ceil(0.5 * n))
    blk = 1024

    idx_pad = jnp.concatenate(
        [obs_x_index.astype(jnp.int32),
         jnp.zeros((kop - ko,), jnp.int32)])
    x_obs = _sc_gather(x, idx_pad)
    gw = _obs_prep(x_obs, W_obs0, b_obs0.reshape(1, h), W_obs1,
                   b_obs1.reshape(1, h), W_obs2, b_obs2.reshape(1, h),
                   W_bil, ko)
    decoded, vmat, score_col = _body(
        x, (W_q0, W_q1, W_q2),
        (b_q0.reshape(1, h), b_q1.reshape(1, h), b_q2.reshape(1, h)),
        (W_v0, W_v1, W_v2),
        (b_v0.reshape(1, h), b_v1.reshape(1, h), b_v2.reshape(1, h)),
        gw.T, b_bil.reshape(1, 2), blk)
    npad = score_col.shape[0]
    score_mat = score_col.reshape(npad // 128, 128)
    pooled, logits = _select_pool(score_mat, score_col, vmat,
                                  W_g, b_g.reshape(1, -1), k_pool)
    return pooled, logits, decoded
